# baseline (speedup 1.0000x reference)
import functools

import jax
import jax.numpy as jnp
from jax import lax
from jax.experimental import pallas as pl
from jax.experimental.pallas import tpu as pltpu
from jax.experimental.pallas import tpu_sc as plsc

L = 10
B = 2
N = 2048
D = 1024
I = 256
NPAIR = B * L
NROWS = NPAIR * I
NW = 32
RPW = NROWS // NW
LANES = 16
DCH = D // LANES

CS = 32
NCH = RPW // CS
NBUF = 3


def _sc_body(table_hbm, idx_hbm, bias_hbm, out_hbm, idx_vm, bias_vm,
             rows_vm, gsem, ssem, bsem):
    wid = lax.axis_index("s") * 2 + lax.axis_index("c")
    base = wid * RPW

    q0 = base // I
    q1 = jnp.minimum(q0 + 1, NPAIR - 1)
    split = (q0 + 1) * I

    pltpu.sync_copy(idx_hbm.at[wid], idx_vm)
    bias_cp = pltpu.async_copy(bias_hbm, bias_vm, bsem)

    def start_gather(c):
        return pltpu.async_copy(table_hbm.at[idx_vm.at[c]],
                                rows_vm.at[c % NBUF], gsem.at[c % NBUF])

    def start_store(c):
        return pltpu.async_copy(rows_vm.at[c % NBUF],
                                out_hbm.at[pl.ds(base + c * CS, CS)],
                                ssem.at[c % NBUF])

    gathers = [None] * NCH
    stores = [None] * NCH
    for c in range(min(NBUF - 1, NCH)):
        gathers[c] = start_gather(c)
    bias_cp.wait()
    for c in range(NCH):
        gathers[c].wait()
        buf = rows_vm.at[c % NBUF]

        @plsc.parallel_loop(0, DCH, unroll=2)
        def _(j):
            sl = pl.ds(j * LANES, LANES)
            bv0 = bias_vm[q0, sl]
            bv1 = bias_vm[q1, sl]
            for r in range(CS):
                bv = jnp.where(base + c * CS + r < split, bv0, bv1)
                buf[r, sl] = buf[r, sl] + bv

        stores[c] = start_store(c)
        if c + NBUF - 1 < NCH:
            if c >= 1:
                stores[c - 1].wait()
            gathers[c + NBUF - 1] = start_gather(c + NBUF - 1)
    for c in range(max(0, NCH - NBUF), NCH):
        stores[c].wait()


@jax.jit
def _run(table, idx_w, bias):
    grid_kernel = functools.partial(
        pl.kernel,
        out_type=jax.ShapeDtypeStruct((NROWS, D), jnp.float32),
        mesh=plsc.VectorSubcoreMesh(core_axis_name="c", subcore_axis_name="s"),
        scratch_types=[
            pltpu.VMEM((NCH, CS), jnp.int32),
            pltpu.VMEM((NPAIR, D), jnp.float32),
            pltpu.VMEM((NBUF, CS, D), jnp.float32),
            pltpu.SemaphoreType.DMA((NBUF,)),
            pltpu.SemaphoreType.DMA((NBUF,)),
            pltpu.SemaphoreType.DMA,
        ],
    )
    return grid_kernel(_sc_body)(table, idx_w, bias)


def kernel(tensor_list, other, index):
    table = tensor_list.reshape(L * B * N, D)

    b_ids = jnp.arange(B, dtype=jnp.int32)
    l_ids = jnp.arange(L, dtype=jnp.int32)
    pair_base = (l_ids[None, :] * B + b_ids[:, None]).reshape(NPAIR) * N
    idx_q = pair_base[:, None] + index[None, :].astype(jnp.int32)
    idx_w = idx_q.reshape(NW, NCH, CS)

    bias = other.reshape(NPAIR, D)

    out = _run(table, idx_w, bias)
    return out.reshape(B, L, I, D)
